# Initial kernel scaffold; baseline (speedup 1.0000x reference)
import jax
import jax.numpy as jnp
from jax.experimental import pallas as pl


def kernel(x, edge_index, batch, W1, b1, W2, b2, W3, b3, Wg1, bg1, Wg2, bg2, Wl1, bl1, Wl2, bl2):
    raise NotImplementedError("write your pallas kernel here")


# speedup vs baseline: 12.1661x; 12.1661x over previous
import functools

import jax
import jax.numpy as jnp
from jax import lax
from jax.experimental import pallas as pl
from jax.experimental.pallas import tpu as pltpu
from jax.experimental.pallas import tpu_sc as plsc

N = 10000
E = 320000
D = 128
H = 128
OUT = 10
G = 64

NC = 2
NS = 16
NW = NC * NS
CH = 128
K = 79
EPT = K * CH
EPAD = NW * EPT
NPAD = 10240
ROWS = NPAD // NS

_sc_mesh = plsc.VectorSubcoreMesh(core_axis_name="c", subcore_axis_name="s")


def _deg_body(dst_hbm, zeros_hbm, d0_hbm, d1_hbm, idx_v, ones_v, acc, sem):
    c = lax.axis_index("c")
    s = lax.axis_index("s")
    wid = c * NS + s
    rb = s * ROWS
    pltpu.sync_copy(zeros_hbm.at[pl.ds(rb, ROWS)], acc.at[pl.ds(rb, ROWS)])
    for i in range(CH // 16):
        ones_v[pl.ds(i * 16, 16)] = jnp.ones((16,), jnp.float32)
    pltpu.sync_copy(dst_hbm.at[wid], idx_v)
    plsc.subcore_barrier()

    def body(j, carry):
        pltpu.sync_copy(ones_v, acc.at[idx_v.at[j]], add=True)
        return carry

    lax.fori_loop(0, K, body, 0)
    plsc.subcore_barrier()

    @pl.when(c == 0)
    def _():
        pltpu.sync_copy(acc.at[pl.ds(rb, ROWS)], d0_hbm.at[pl.ds(rb, ROWS)])

    @pl.when(c == 1)
    def _():
        pltpu.sync_copy(acc.at[pl.ds(rb, ROWS)], d1_hbm.at[pl.ds(rb, ROWS)])


_deg_call = functools.partial(
    pl.kernel,
    out_type=[
        jax.ShapeDtypeStruct((NPAD,), jnp.float32),
        jax.ShapeDtypeStruct((NPAD,), jnp.float32),
    ],
    mesh=_sc_mesh,
    scratch_types=[
        pltpu.VMEM((K, CH), jnp.int32),
        pltpu.VMEM((CH,), jnp.float32),
        pltpu.VMEM_SHARED((NPAD,), jnp.float32),
        pltpu.SemaphoreType.DMA,
    ],
)(_deg_body)


def _agg_body(src_hbm, dst_hbm, hs_hbm, p0_hbm, p1_hbm,
              sidx, didx, rows, acc, sem):
    c = lax.axis_index("c")
    s = lax.axis_index("s")
    wid = c * NS + s
    rb = s * ROWS
    pltpu.sync_copy(hs_hbm.at[pl.ds(rb, ROWS)], acc.at[pl.ds(rb, ROWS)])
    pltpu.sync_copy(src_hbm.at[wid], sidx)
    pltpu.sync_copy(dst_hbm.at[wid], didx)
    plsc.subcore_barrier()

    def body(j, carry):
        pltpu.async_copy(hs_hbm.at[sidx.at[j]], rows, sem).wait()
        pltpu.sync_copy(rows, acc.at[didx.at[j]], add=True)
        return carry

    lax.fori_loop(0, K, body, 0)
    plsc.subcore_barrier()

    @pl.when(c == 0)
    def _():
        pltpu.sync_copy(acc.at[pl.ds(rb, ROWS)], p0_hbm.at[pl.ds(rb, ROWS)])

    @pl.when(c == 1)
    def _():
        pltpu.sync_copy(acc.at[pl.ds(rb, ROWS)], p1_hbm.at[pl.ds(rb, ROWS)])


_agg_call = functools.partial(
    pl.kernel,
    out_type=[
        jax.ShapeDtypeStruct((NPAD, H), jnp.float32),
        jax.ShapeDtypeStruct((NPAD, H), jnp.float32),
    ],
    mesh=_sc_mesh,
    scratch_types=[
        pltpu.VMEM((K, CH), jnp.int32),
        pltpu.VMEM((K, CH), jnp.int32),
        pltpu.VMEM((CH, H), jnp.float32),
        pltpu.VMEM_SHARED((NPAD, H), jnp.float32),
        pltpu.SemaphoreType.DMA,
    ],
)(_agg_body)


def _tc_first_body(d0_ref, d1_ref, x_ref, w1_ref, dis_ref, hs_ref):
    deg = d0_ref[...] + d1_ref[...] + 1.0
    dis = lax.rsqrt(deg)
    dis_ref[...] = dis
    hs_ref[...] = jnp.dot(x_ref[...], w1_ref[...],
                          preferred_element_type=jnp.float32) * dis


_tc_first = pl.pallas_call(
    _tc_first_body,
    out_shape=[
        jax.ShapeDtypeStruct((NPAD, 1), jnp.float32),
        jax.ShapeDtypeStruct((NPAD, H), jnp.float32),
    ],
)


def _tc_mid_body(p0_ref, p1_ref, hs_ref, dis_ref, b_ref, w_ref, hsn_ref):
    dis = dis_ref[...]
    agg = p0_ref[...] + p1_ref[...] - hs_ref[...]
    h = jnp.maximum(agg * dis + b_ref[...], 0.0)
    hsn_ref[...] = jnp.dot(h, w_ref[...],
                           preferred_element_type=jnp.float32) * dis


_tc_mid = pl.pallas_call(
    _tc_mid_body,
    out_shape=jax.ShapeDtypeStruct((NPAD, H), jnp.float32),
)


def _tc_final_body(p0_ref, p1_ref, hs_ref, dis_ref, b_ref,
                   wg1_ref, bg1_ref, wg2_ref, bg2_ref,
                   wl1_ref, bl1_ref, wl2_ref, bl2_ref,
                   batch_ref, logits_ref):
    agg = p0_ref[...] + p1_ref[...] - hs_ref[...]
    h_all = jnp.maximum(agg * dis_ref[...] + b_ref[...], 0.0)
    h = h_all[:N, :]
    u = jnp.maximum(jnp.dot(h, wg1_ref[...],
                            preferred_element_type=jnp.float32) + bg1_ref[...],
                    0.0)
    gate = jnp.dot(u, wg2_ref[...],
                   preferred_element_type=jnp.float32) + bg2_ref[...]
    gid = lax.broadcasted_iota(jnp.int32, (N, G), 1)
    msk = gid == batch_ref[...]
    mskf = msk.astype(jnp.float32)
    neg = jnp.full((N, G), -jnp.inf, jnp.float32)
    m = jnp.max(jnp.where(msk, gate, neg), axis=0, keepdims=True)
    m = jnp.where(jnp.isfinite(m), m, 0.0)
    msel = jnp.sum(mskf * m, axis=1, keepdims=True)
    e = jnp.exp(gate - msel)
    denom = jnp.sum(mskf * e, axis=0, keepdims=True)
    densel = jnp.sum(mskf * denom, axis=1, keepdims=True)
    alpha = e / jnp.maximum(densel, 1e-16)
    pooled = lax.dot_general(mskf * alpha, h,
                             dimension_numbers=(((0,), (0,)), ((), ())),
                             preferred_element_type=jnp.float32)
    z = jnp.maximum(jnp.dot(pooled, wl1_ref[...],
                            preferred_element_type=jnp.float32) + bl1_ref[...],
                    0.0)
    logits_ref[...] = jnp.dot(z, wl2_ref[...],
                              preferred_element_type=jnp.float32) + bl2_ref[...]


_tc_final = pl.pallas_call(
    _tc_final_body,
    out_shape=jax.ShapeDtypeStruct((G, OUT), jnp.float32),
)


@jax.jit
def kernel(x, edge_index, batch, W1, b1, W2, b2, W3, b3,
           Wg1, bg1, Wg2, bg2, Wl1, bl1, Wl2, bl2):
    src = edge_index[0]
    dst = edge_index[1]
    pad = EPAD - E
    src_p = jnp.concatenate([src, jnp.zeros((pad,), jnp.int32)]
                            ).reshape(NW, K, CH)
    dst_p = jnp.concatenate([dst, jnp.full((pad,), N, jnp.int32)]
                            ).reshape(NW, K, CH)
    x_p = jnp.pad(x, ((0, NPAD - N), (0, 0)))
    zeros_n = jnp.zeros((NPAD,), jnp.float32)
    batch_col = batch.reshape(N, 1)

    d0, d1 = _deg_call(dst_p, zeros_n)
    dis, hs = _tc_first(d0.reshape(NPAD, 1), d1.reshape(NPAD, 1), x_p, W1)

    p0, p1 = _agg_call(src_p, dst_p, hs)
    hs2 = _tc_mid(p0, p1, hs, dis, b1.reshape(1, H), W2)
    p0, p1 = _agg_call(src_p, dst_p, hs2)
    hs3 = _tc_mid(p0, p1, hs2, dis, b2.reshape(1, H), W3)
    p0, p1 = _agg_call(src_p, dst_p, hs3)

    logits = _tc_final(p0, p1, hs3, dis, b3.reshape(1, H),
                       Wg1, bg1.reshape(1, H), Wg2, bg2.reshape(1, 1),
                       Wl1, bl1.reshape(1, H), Wl2, bl2.reshape(1, OUT),
                       batch_col)
    return (logits, jnp.zeros((), jnp.float32))
